# baseline (speedup 1.0000x reference)
import functools

import jax
import jax.numpy as jnp
from jax import lax
from jax.experimental import pallas as pl
from jax.experimental.pallas import tpu as pltpu
from jax.experimental.pallas import tpu_sc as plsc

IN_F = 4096
OUT_F = 4096
GS = 8
GROUPS = IN_F // GS
NCB = 2
CB_SIZE = 2 ** 16
IDX_PER_ROW = GROUPS * NCB
NCHUNK = IDX_PER_ROW // 128
NC, NS = 2, 16
NW = NC * NS
ROWS_PER_W = OUT_F // NW
RB = OUT_F // 8
KB = IN_F // 128


def _sc_gather_body(
    table_hbm, idx_hbm, w4_hbm, idxb3, idxb, rows, ws, table_sp, isem, gsem, wsem
):
    sid = lax.axis_index("s")
    wid = sid * NC + lax.axis_index("c")
    base_row = wid * ROWS_PER_W

    @pl.when(sid == 0)
    def _():
        pltpu.sync_copy(table_hbm, table_sp)

    plsc.subcore_barrier()
    lane = lax.iota(jnp.int32, 16)
    cb_off = (lane & 1) << 16
    half = lane >> 3
    lanemod = lane & 7

    def add_offsets(slot):
        def off_body(j, carry3):
            idxb[slot, pl.ds(j * 16, 16)] = (
                idxb3[slot, j >> 3, pl.ds((j & 7) * 16, 16)] + cb_off
            )
            return carry3

        lax.fori_loop(0, IDX_PER_ROW // 16, off_body, 0, unroll=8)

    def fire_gather(slot):
        pltpu.async_copy(table_sp.at[idxb.at[slot]], rows.at[slot], gsem)

    for r0 in range(3):
        pltpu.sync_copy(idx_hbm.at[base_row + r0], idxb3.at[r0])
        add_offsets(r0)
        fire_gather(r0)
    pltpu.async_copy(idx_hbm.at[base_row + 3], idxb3.at[3], isem)

    def row_body(r, carry):
        o = base_row + r
        p = r & 3
        n3 = (r + 3) & 3
        pw = r & 1
        pltpu.make_async_copy(table_sp.at[idxb.at[p]], rows.at[p], gsem).wait()

        @pl.when(r + 4 < ROWS_PER_W)
        def _():
            pltpu.async_copy(idx_hbm.at[o + 4], idxb3.at[p], isem)

        @pl.when(r + 3 < ROWS_PER_W)
        def _():
            pltpu.make_async_copy(
                idx_hbm.at[o + 3], idxb3.at[n3], isem
            ).wait()
            add_offsets(n3)
            fire_gather(n3)

        @pl.when(r >= 2)
        def _():
            pltpu.make_async_copy(
                ws.at[pw], w4_hbm.at[(o - 2) // 8, :, (o - 2) % 8, :], wsem
            ).wait()

        pvec = lane * 0 + p

        def sum_body(t, carry2):
            base = 4 * t + 2 * half
            a = plsc.load_gather(rows, [pvec, base, lanemod])
            b = plsc.load_gather(rows, [pvec, base + 1, lanemod])
            ws[pw, t >> 3, pl.ds((t & 7) * 16, 16)] = a + b
            return carry2

        lax.fori_loop(0, IN_F // 16, sum_body, 0, unroll=8)
        pltpu.async_copy(ws.at[pw], w4_hbm.at[o // 8, :, o % 8, :], wsem)
        return carry

    lax.fori_loop(0, ROWS_PER_W, row_body, 0)

    last = base_row + ROWS_PER_W - 2
    pltpu.make_async_copy(
        ws.at[0], w4_hbm.at[last // 8, :, last % 8, :], wsem
    ).wait()
    pltpu.make_async_copy(
        ws.at[1], w4_hbm.at[(last + 1) // 8, :, (last + 1) % 8, :], wsem
    ).wait()


@jax.jit
def _sc_gather(table, idx):
    mesh = plsc.VectorSubcoreMesh(core_axis_name="c", subcore_axis_name="s")
    f = functools.partial(
        pl.kernel,
        out_type=jax.ShapeDtypeStruct((RB, KB, 8, 128), jnp.float32),
        mesh=mesh,
        scratch_types=[
            pltpu.VMEM((4, NCHUNK, 128), jnp.int32),
            pltpu.VMEM((4, IDX_PER_ROW), jnp.int32),
            pltpu.VMEM((4, IDX_PER_ROW, GS), jnp.float32),
            pltpu.VMEM((2, KB, 128), jnp.float32),
            pltpu.VMEM_SHARED((NCB * CB_SIZE, GS), jnp.float32),
            pltpu.SemaphoreType.DMA,
            pltpu.SemaphoreType.DMA,
            pltpu.SemaphoreType.DMA,
        ],
        compiler_params=pltpu.CompilerParams(
            use_tc_tiling_on_sc=False, needs_layout_passes=False
        ),
    )(_sc_gather_body)
    return f(table, idx)


def _mm_body(x_ref, w_ref, s_ref, b_ref, o_ref):
    acc = jnp.zeros((32, w_ref.shape[0] * 8), jnp.float32)
    for kb in range(KB):
        w2 = w_ref[:, kb].reshape(w_ref.shape[0] * 8, 128)
        acc = acc + lax.dot_general(
            x_ref[:, pl.ds(kb * 128, 128)],
            w2,
            (((1,), (1,)), ((), ())),
            preferred_element_type=jnp.float32,
        )
    o_ref[...] = acc * s_ref[...] + b_ref[...]


@jax.jit
def _tc_matmul(x, w4, scales_row, bias_row):
    rbb = 64
    grid = (RB // rbb,)
    return pl.pallas_call(
        _mm_body,
        grid=grid,
        in_specs=[
            pl.BlockSpec((32, IN_F), lambda j: (0, 0)),
            pl.BlockSpec((rbb, KB, 8, 128), lambda j: (j, 0, 0, 0)),
            pl.BlockSpec((1, rbb * 8), lambda j: (0, j)),
            pl.BlockSpec((1, rbb * 8), lambda j: (0, j)),
        ],
        out_specs=pl.BlockSpec((32, rbb * 8), lambda j: (0, j)),
        out_shape=jax.ShapeDtypeStruct((32, OUT_F), jnp.float32),
        compiler_params=pltpu.CompilerParams(
            dimension_semantics=("arbitrary",)
        ),
    )(x, w4, scales_row, bias_row)


def kernel(input, codes, codebooks, scales, bias):
    table = codebooks.reshape(NCB * CB_SIZE, GS)
    idx = codes.reshape(OUT_F, NCHUNK, 128)
    w4 = _sc_gather(table, idx)
    return _tc_matmul(
        input, w4, scales.reshape(1, OUT_F), bias.reshape(1, OUT_F)
    )


# speedup vs baseline: 1.0127x; 1.0127x over previous
import functools

import jax
import jax.numpy as jnp
from jax import lax
from jax.experimental import pallas as pl
from jax.experimental.pallas import tpu as pltpu
from jax.experimental.pallas import tpu_sc as plsc

IN_F = 4096
OUT_F = 4096
GS = 8
GROUPS = IN_F // GS
NCB = 2
CB_SIZE = 2 ** 16
IDX_PER_ROW = GROUPS * NCB
NC, NS = 2, 16
NW = NC * NS
RB = OUT_F // 8
KB = IN_F // 128
HALF = OUT_F // 2
HALF_ROWS = HALF // NW


def _make_sc_body(h):
    def _sc_gather_body(
        table_hbm, idx_hbm, w4_hbm, idxb, rows, ws, table_sp, isem, gsem, wsem
    ):
        sid = lax.axis_index("s")
        wid = sid * NC + lax.axis_index("c")
        base_row = h * HALF + wid * HALF_ROWS
        lane = lax.iota(jnp.int32, 16)
        cb_off = (lane & 1) << 16
        half = lane >> 3
        lanemod = lane & 7

        @pl.when(sid == 0)
        def _():
            pltpu.sync_copy(table_hbm, table_sp)

        plsc.subcore_barrier()

        def add_offsets(slot):
            def off_body(j, carry3):
                idxb[slot, pl.ds(j * 16, 16)] = (
                    idxb[slot, pl.ds(j * 16, 16)] + cb_off
                )
                return carry3

            lax.fori_loop(0, IDX_PER_ROW // 16, off_body, 0, unroll=8)

        def fire_gather(slot):
            pltpu.async_copy(table_sp.at[idxb.at[slot]], rows.at[slot], gsem)

        for r0 in range(3):
            pltpu.sync_copy(idx_hbm.at[base_row + r0], idxb.at[r0])
            add_offsets(r0)
            fire_gather(r0)
        pltpu.async_copy(idx_hbm.at[base_row + 3], idxb.at[3], isem)

        def row_body(r, carry):
            o = base_row + r
            ob = o - h * HALF
            p = r & 3
            n3 = (r + 3) & 3
            pw = r & 1
            pltpu.make_async_copy(
                table_sp.at[idxb.at[p]], rows.at[p], gsem
            ).wait()

            @pl.when(r + 4 < HALF_ROWS)
            def _():
                pltpu.async_copy(idx_hbm.at[o + 4], idxb.at[p], isem)

            @pl.when(r + 3 < HALF_ROWS)
            def _():
                pltpu.make_async_copy(
                    idx_hbm.at[o + 3], idxb.at[n3], isem
                ).wait()
                add_offsets(n3)
                fire_gather(n3)

            @pl.when(r >= 2)
            def _():
                pltpu.make_async_copy(
                    ws.at[pw],
                    w4_hbm.at[(ob - 2) // 8, :, (ob - 2) % 8, :],
                    wsem,
                ).wait()

            pvec = lane * 0 + p

            def sum_body(t, carry2):
                base = 4 * t + 2 * half
                a = plsc.load_gather(rows, [pvec, base, lanemod])
                b = plsc.load_gather(rows, [pvec, base + 1, lanemod])
                ws[pw, t >> 3, pl.ds((t & 7) * 16, 16)] = a + b
                return carry2

            lax.fori_loop(0, IN_F // 16, sum_body, 0, unroll=8)
            pltpu.async_copy(
                ws.at[pw], w4_hbm.at[ob // 8, :, ob % 8, :], wsem
            )
            return carry

        lax.fori_loop(0, HALF_ROWS, row_body, 0)

        lastb = wid * HALF_ROWS + HALF_ROWS - 2
        pltpu.make_async_copy(
            ws.at[0], w4_hbm.at[lastb // 8, :, lastb % 8, :], wsem
        ).wait()
        pltpu.make_async_copy(
            ws.at[1], w4_hbm.at[(lastb + 1) // 8, :, (lastb + 1) % 8, :], wsem
        ).wait()

    return _sc_gather_body


def _sc_gather_half(table, idx, h):
    mesh = plsc.VectorSubcoreMesh(core_axis_name="c", subcore_axis_name="s")
    f = functools.partial(
        pl.kernel,
        out_type=jax.ShapeDtypeStruct((RB // 2, KB, 8, 128), jnp.float32),
        mesh=mesh,
        scratch_types=[
            pltpu.VMEM((4, IDX_PER_ROW), jnp.int32),
            pltpu.VMEM((4, IDX_PER_ROW, GS), jnp.float32),
            pltpu.VMEM((2, KB, 128), jnp.float32),
            pltpu.VMEM_SHARED((NCB * CB_SIZE, GS), jnp.float32),
            pltpu.SemaphoreType.DMA,
            pltpu.SemaphoreType.DMA,
            pltpu.SemaphoreType.DMA,
        ],
        compiler_params=pltpu.CompilerParams(
            use_tc_tiling_on_sc=False, needs_layout_passes=False
        ),
        name=f"sc_dequant_h{h}",
    )(_make_sc_body(h))
    return f(table, idx)


def _mm_body(x_ref, w_ref, s_ref, b_ref, o_ref):
    acc = jnp.zeros((32, w_ref.shape[0] * 8), jnp.float32)
    for kb in range(KB):
        w2 = w_ref[:, kb].reshape(w_ref.shape[0] * 8, 128)
        acc = acc + lax.dot_general(
            x_ref[:, pl.ds(kb * 128, 128)],
            w2,
            (((1,), (1,)), ((), ())),
            preferred_element_type=jnp.float32,
        )
    o_ref[...] = acc * s_ref[...] + b_ref[...]


def _tc_matmul_half(x, w4, scales_row, bias_row):
    rbb = 64
    grid = (RB // 2 // rbb,)
    return pl.pallas_call(
        _mm_body,
        grid=grid,
        in_specs=[
            pl.BlockSpec((32, IN_F), lambda j: (0, 0)),
            pl.BlockSpec((rbb, KB, 8, 128), lambda j: (j, 0, 0, 0)),
            pl.BlockSpec((1, rbb * 8), lambda j: (0, j)),
            pl.BlockSpec((1, rbb * 8), lambda j: (0, j)),
        ],
        out_specs=pl.BlockSpec((32, rbb * 8), lambda j: (0, j)),
        out_shape=jax.ShapeDtypeStruct((32, HALF), jnp.float32),
        compiler_params=pltpu.CompilerParams(
            dimension_semantics=("arbitrary",)
        ),
    )(x, w4, scales_row, bias_row)


@jax.jit
def _run(input, table, idx, scales_row, bias_row):
    w4a = _sc_gather_half(table, idx, 0)
    w4b = _sc_gather_half(table, idx, 1)
    outa = _tc_matmul_half(
        input, w4a, scales_row[:, :HALF], bias_row[:, :HALF]
    )
    outb = _tc_matmul_half(
        input, w4b, scales_row[:, HALF:], bias_row[:, HALF:]
    )
    return jnp.concatenate([outa, outb], axis=1)


def kernel(input, codes, codebooks, scales, bias):
    table = codebooks.reshape(NCB * CB_SIZE, GS)
    idx = codes.reshape(OUT_F, IDX_PER_ROW)
    return _run(
        input, table, idx, scales.reshape(1, OUT_F), bias.reshape(1, OUT_F)
    )


# speedup vs baseline: 1.0323x; 1.0193x over previous
import functools

import jax
import jax.numpy as jnp
from jax import lax
from jax.experimental import pallas as pl
from jax.experimental.pallas import tpu as pltpu
from jax.experimental.pallas import tpu_sc as plsc

IN_F = 4096
OUT_F = 4096
GS = 8
GROUPS = IN_F // GS
NCB = 2
CB_SIZE = 2 ** 16
IDX_PER_ROW = GROUPS * NCB
NCHUNK = IDX_PER_ROW // 128
NC, NS = 2, 16
NW = NC * NS
ROWS_PER_W = OUT_F // NW
RB = OUT_F // 8
KB = IN_F // 128


def _sc_gather_body(
    table_hbm, idx_hbm, w4_hbm, idxb, rows, ws, table_sp, isem, gsem, wsem
):
    sid = lax.axis_index("s")
    wid = sid * NC + lax.axis_index("c")
    base_row = wid * ROWS_PER_W

    @pl.when(sid == 0)
    def _():
        pltpu.sync_copy(table_hbm, table_sp)

    plsc.subcore_barrier()
    lane = lax.iota(jnp.int32, 16)
    cb_off = (lane & 1) << 16
    half = lane >> 3
    lanemod = lane & 7

    def add_offsets(slot):
        def off_body(j, carry3):
            idxb[slot, pl.ds(j * 16, 16)] = (
                idxb[slot, pl.ds(j * 16, 16)] + cb_off
            )
            return carry3

        lax.fori_loop(0, IDX_PER_ROW // 16, off_body, 0, unroll=8)

    def fire_gather(slot):
        pltpu.async_copy(table_sp.at[idxb.at[slot]], rows.at[slot], gsem)

    for r0 in range(3):
        pltpu.sync_copy(idx_hbm.at[base_row + r0], idxb.at[r0])
        add_offsets(r0)
        fire_gather(r0)
    pltpu.async_copy(idx_hbm.at[base_row + 3], idxb.at[3], isem)

    def row_body(r, carry):
        o = base_row + r
        p = r & 3
        n3 = (r + 3) & 3
        pw = r & 1
        pltpu.make_async_copy(table_sp.at[idxb.at[p]], rows.at[p], gsem).wait()

        @pl.when(r + 4 < ROWS_PER_W)
        def _():
            pltpu.async_copy(idx_hbm.at[o + 4], idxb.at[p], isem)

        @pl.when(r + 3 < ROWS_PER_W)
        def _():
            pltpu.make_async_copy(
                idx_hbm.at[o + 3], idxb.at[n3], isem
            ).wait()
            add_offsets(n3)
            fire_gather(n3)

        @pl.when(r >= 2)
        def _():
            pltpu.make_async_copy(
                ws.at[pw], w4_hbm.at[(o - 2) // 8, :, (o - 2) % 8, :], wsem
            ).wait()

        pvec = lane * 0 + p

        def sum_body(t, carry2):
            base = 4 * t + 2 * half
            a = plsc.load_gather(rows, [pvec, base, lanemod])
            b = plsc.load_gather(rows, [pvec, base + 1, lanemod])
            ws[pw, t >> 3, pl.ds((t & 7) * 16, 16)] = a + b
            return carry2

        lax.fori_loop(0, IN_F // 16, sum_body, 0, unroll=8)
        pltpu.async_copy(ws.at[pw], w4_hbm.at[o // 8, :, o % 8, :], wsem)
        return carry

    lax.fori_loop(0, ROWS_PER_W, row_body, 0)

    last = base_row + ROWS_PER_W - 2
    pltpu.make_async_copy(
        ws.at[0], w4_hbm.at[last // 8, :, last % 8, :], wsem
    ).wait()
    pltpu.make_async_copy(
        ws.at[1], w4_hbm.at[(last + 1) // 8, :, (last + 1) % 8, :], wsem
    ).wait()


@jax.jit
def _sc_gather(table, idx):
    mesh = plsc.VectorSubcoreMesh(core_axis_name="c", subcore_axis_name="s")
    f = functools.partial(
        pl.kernel,
        out_type=jax.ShapeDtypeStruct((RB, KB, 8, 128), jnp.float32),
        mesh=mesh,
        scratch_types=[
            pltpu.VMEM((4, IDX_PER_ROW), jnp.int32),
            pltpu.VMEM((4, IDX_PER_ROW, GS), jnp.float32),
            pltpu.VMEM((2, KB, 128), jnp.float32),
            pltpu.VMEM_SHARED((NCB * CB_SIZE, GS), jnp.float32),
            pltpu.SemaphoreType.DMA,
            pltpu.SemaphoreType.DMA,
            pltpu.SemaphoreType.DMA,
        ],
        compiler_params=pltpu.CompilerParams(
            use_tc_tiling_on_sc=False, needs_layout_passes=False
        ),
    )(_sc_gather_body)
    return f(table, idx)


def _mm_body(x_ref, w_ref, s_ref, b_ref, o_ref):
    acc = jnp.zeros((32, w_ref.shape[0] * 8), jnp.float32)
    for kb in range(KB):
        w2 = w_ref[:, kb].reshape(w_ref.shape[0] * 8, 128)
        acc = acc + lax.dot_general(
            x_ref[:, pl.ds(kb * 128, 128)],
            w2,
            (((1,), (1,)), ((), ())),
            preferred_element_type=jnp.float32,
        )
    o_ref[...] = acc * s_ref[...] + b_ref[...]


@jax.jit
def _tc_matmul(x, w4, scales_row, bias_row):
    rbb = 64
    grid = (RB // rbb,)
    return pl.pallas_call(
        _mm_body,
        grid=grid,
        in_specs=[
            pl.BlockSpec((32, IN_F), lambda j: (0, 0)),
            pl.BlockSpec((rbb, KB, 8, 128), lambda j: (j, 0, 0, 0)),
            pl.BlockSpec((1, rbb * 8), lambda j: (0, j)),
            pl.BlockSpec((1, rbb * 8), lambda j: (0, j)),
        ],
        out_specs=pl.BlockSpec((32, rbb * 8), lambda j: (0, j)),
        out_shape=jax.ShapeDtypeStruct((32, OUT_F), jnp.float32),
        compiler_params=pltpu.CompilerParams(
            dimension_semantics=("arbitrary",)
        ),
    )(x, w4, scales_row, bias_row)


def kernel(input, codes, codebooks, scales, bias):
    table = codebooks.reshape(NCB * CB_SIZE, GS)
    idx = codes.reshape(OUT_F, IDX_PER_ROW)
    w4 = _sc_gather(table, idx)
    return _tc_matmul(
        input, w4, scales.reshape(1, OUT_F), bias.reshape(1, OUT_F)
    )
